# Initial kernel scaffold; baseline (speedup 1.0000x reference)
import jax
import jax.numpy as jnp
from jax.experimental import pallas as pl


def kernel(R, params, Z, BN_edge_index):
    raise NotImplementedError("write your pallas kernel here")


# speedup vs baseline: 1.4936x; 1.4936x over previous
import functools
import math

import jax
import jax.numpy as jnp
from jax import lax
from jax.experimental import pallas as pl
from jax.experimental.pallas import tpu as pltpu
from jax.experimental.pallas import tpu_sc as plsc

N = 10000
E = 160000
F = 256
K = 64
LOG2 = math.log(2.0)

NB = 10
BN = N // NB
EB = 80
BE = E // EB

NW = 32
GCH = 40
SCH = 80
HALF = F // 2


def _ssp(x):
    return jax.nn.softplus(x) - LOG2


def _sc_gather(D):
    per_w = E // NW
    iters = per_w // GCH
    mesh = plsc.VectorSubcoreMesh(core_axis_name="c", subcore_axis_name="s")

    @functools.partial(
        pl.kernel, mesh=mesh,
        out_type=jax.ShapeDtypeStruct((E, D), jnp.float32),
        scratch_types=[
            pltpu.VMEM((per_w,), jnp.int32),
            pltpu.VMEM((GCH, D), jnp.float32),
            pltpu.SemaphoreType.DMA,
        ],
    )
    def k(table_hbm, idx_hbm, out_hbm, idx_v, rows_v, sem):
        wid = lax.axis_index("s") * 2 + lax.axis_index("c")
        base = wid * per_w
        pltpu.sync_copy(idx_hbm.at[pl.ds(base, per_w)], idx_v)

        def body(i, carry):
            off = i * GCH
            pltpu.async_copy(
                table_hbm.at[idx_v.at[pl.ds(off, GCH)]], rows_v, sem
            ).wait()
            pltpu.sync_copy(rows_v, out_hbm.at[pl.ds(base + off, GCH)])
            return carry

        lax.fori_loop(0, iters, body, 0)

    return k


NROW = 5120
NACC = NROW + 8
ZR = NROW // 16


def _sc_scatter_half(hoff):
    per_s = E // 16
    iters = per_s // SCH
    mesh = plsc.VectorSubcoreMesh(core_axis_name="c", subcore_axis_name="s")

    @functools.partial(
        pl.kernel, mesh=mesh,
        out_type=jax.ShapeDtypeStruct((2 * NROW, HALF), jnp.float32),
        scratch_types=[
            pltpu.VMEM((SCH,), jnp.int32),
            pltpu.VMEM((SCH, HALF), jnp.float32),
            pltpu.VMEM((ZR, HALF), jnp.float32),
            pltpu.VMEM_SHARED((NACC, HALF), jnp.float32),
            pltpu.SemaphoreType.DMA,
        ],
    )
    def k(msg_hbm, dst_hbm, zero_hbm, agg_hbm, idx_v, rows_v, stripe_v,
          shared, sem):
        c = lax.axis_index("c")
        s = lax.axis_index("s")
        nbase = c * NROW
        pltpu.sync_copy(zero_hbm, stripe_v)
        pltpu.sync_copy(stripe_v, shared.at[pl.ds(s * ZR, ZR)])
        plsc.subcore_barrier()

        def body(i, carry):
            off = s * per_s + i * SCH
            pltpu.sync_copy(dst_hbm.at[pl.ds(off, SCH)], idx_v)
            pltpu.sync_copy(
                msg_hbm.at[pl.ds(off, SCH), pl.ds(hoff, HALF)], rows_v)
            for j in range(SCH // 16):
                v = idx_v[pl.ds(j * 16, 16)]
                rel = v - nbase
                ok = (rel >= 0) & (rel < NROW)
                idx_v[pl.ds(j * 16, 16)] = jnp.where(ok, rel, NROW)
            pltpu.sync_copy(rows_v, shared.at[idx_v], add=True)
            return carry

        lax.fori_loop(0, iters, body, 0)
        plsc.subcore_barrier()
        pltpu.sync_copy(shared.at[pl.ds(s * ZR, ZR)], stripe_v)
        pltpu.sync_copy(stripe_v, agg_hbm.at[pl.ds(nbase + s * ZR, ZR)])

    return k


def _emb_body(z_ref, emb_ref, o_ref):
    col = lax.broadcasted_iota(jnp.int32, (BN, 128), 1)
    oh = (col == z_ref[:, :1]).astype(jnp.float32)
    o_ref[...] = jnp.dot(oh, emb_ref[...], preferred_element_type=jnp.float32)


def _tc_emb(zcol, emb_pad):
    return pl.pallas_call(
        _emb_body,
        grid=(NB,),
        in_specs=[pl.BlockSpec((BN, 1), lambda i: (i, 0)),
                  pl.BlockSpec((128, F), lambda i: (0, 0))],
        out_specs=pl.BlockSpec((BN, F), lambda i: (i, 0)),
        out_shape=jax.ShapeDtypeStruct((N, F), jnp.float32),
    )(zcol, emb_pad)


def _nodeA_body(x_ref, wj_ref, bj_ref, wi_ref, bi_ref, xj_ref, xi_ref):
    xa = _ssp(x_ref[...])
    xj_ref[...] = jnp.dot(xa, wj_ref[...],
                          preferred_element_type=jnp.float32) + bj_ref[...]
    xi_ref[...] = jnp.dot(xa, wi_ref[...],
                          preferred_element_type=jnp.float32) + bi_ref[...]


def _tc_nodeA(x, Wj, bj, Wi, bi):
    w = pl.BlockSpec((F, F), lambda i: (0, 0))
    b = pl.BlockSpec((1, F), lambda i: (0, 0))
    n = pl.BlockSpec((BN, F), lambda i: (i, 0))
    return pl.pallas_call(
        _nodeA_body,
        grid=(NB,),
        in_specs=[n, w, b, w, b],
        out_specs=[n, n],
        out_shape=[jax.ShapeDtypeStruct((N, F), jnp.float32),
                   jax.ShapeDtypeStruct((N, F), jnp.float32)],
    )(x, Wj, bj, Wi, bi)


RD = 128


def _rbf_body(rs_ref, rd_ref, c_ref, w_ref, co_ref, o_ref):
    diff = rs_ref[...] - rd_ref[...]
    d2 = jnp.sum(diff * diff, axis=1, keepdims=True)
    d = jnp.sqrt(d2 + 1e-10)
    cutoff = co_ref[0, 0]
    r = d / cutoff
    phi = 1.0 - 6.0 * r ** 5 + 15.0 * r ** 4 - 10.0 * r ** 3
    phi = jnp.where(d < cutoff, phi, jnp.zeros_like(phi))
    o_ref[...] = phi * jnp.exp(-w_ref[...] * (jnp.exp(-d) - c_ref[...]) ** 2)


def _tc_rbf(rsrc, rdst, centers, widths, cutoff):
    e = pl.BlockSpec((BE, RD), lambda i: (i, 0))
    v = pl.BlockSpec((1, K), lambda i: (0, 0))
    return pl.pallas_call(
        _rbf_body,
        grid=(EB,),
        in_specs=[e, e, v, v, pl.BlockSpec((1, 1), lambda i: (0, 0))],
        out_specs=pl.BlockSpec((BE, K), lambda i: (i, 0)),
        out_shape=jax.ShapeDtypeStruct((E, K), jnp.float32),
    )(rsrc, rdst, centers, widths, cutoff)


def _msg_body(rbf_ref, xjs_ref, g_ref, o_ref):
    g = jnp.dot(rbf_ref[...], g_ref[...], preferred_element_type=jnp.float32)
    o_ref[...] = g * xjs_ref[...]


def _tc_msg(rbf, xjsrc, G):
    return pl.pallas_call(
        _msg_body,
        grid=(EB,),
        in_specs=[pl.BlockSpec((BE, K), lambda i: (i, 0)),
                  pl.BlockSpec((BE, F), lambda i: (i, 0)),
                  pl.BlockSpec((K, F), lambda i: (0, 0))],
        out_specs=pl.BlockSpec((BE, F), lambda i: (i, 0)),
        out_shape=jax.ShapeDtypeStruct((E, F), jnp.float32),
    )(rbf, xjsrc, G)


def _res(x, w1, b1, w2, b2):
    y = jnp.dot(_ssp(x), w1, preferred_element_type=jnp.float32) + b1
    y = jnp.dot(_ssp(y), w2, preferred_element_type=jnp.float32) + b2
    return x + y


def _nodeF_body(x_ref, xi_ref, agg_ref, w1, b1, w2, b2, wo, bo, u,
                a1, ab1, a2, ab2, o_ref):
    mi = xi_ref[...] + agg_ref[...]
    mi = _res(mi, w1[...], b1[...], w2[...], b2[...])
    xn = u[...] * x_ref[...] + jnp.dot(
        _ssp(mi), wo[...], preferred_element_type=jnp.float32) + bo[...]
    o_ref[...] = _res(xn, a1[...], ab1[...], a2[...], ab2[...])


def _tc_nodeF(x, xi, agg, W1, b1, W2, b2, Wo, bo, u, aW1, ab1, aW2, ab2):
    w = pl.BlockSpec((F, F), lambda i: (0, 0))
    b = pl.BlockSpec((1, F), lambda i: (0, 0))
    n = pl.BlockSpec((BN, F), lambda i: (i, 0))
    return pl.pallas_call(
        _nodeF_body,
        grid=(NB,),
        in_specs=[n, n, n, w, b, w, b, w, b, b, w, b, w, b],
        out_specs=n,
        out_shape=jax.ShapeDtypeStruct((N, F), jnp.float32),
    )(x, xi, agg, W1, b1, W2, b2, Wo, bo, u, aW1, ab1, aW2, ab2)


def _out_body(x_ref, w1, b1, w2, b2, ow, z_ref, sc_ref, sh_ref, o_ref):
    o = _res(x_ref[...], w1[...], b1[...], w2[...], b2[...])
    val = jnp.dot(_ssp(o), ow[...], preferred_element_type=jnp.float32)
    col = lax.broadcasted_iota(jnp.int32, (BN, 128), 1)
    oh = (col == z_ref[:, :1]).astype(jnp.float32)
    scale = jnp.dot(oh, sc_ref[...], preferred_element_type=jnp.float32)
    shift = jnp.dot(oh, sh_ref[...], preferred_element_type=jnp.float32)
    o_ref[...] = scale * val + shift


def _tc_out(x, W1, b1, W2, b2, outw_pad, zcol, scale_b, shift_b):
    w = pl.BlockSpec((F, F), lambda i: (0, 0))
    b = pl.BlockSpec((1, F), lambda i: (0, 0))
    n = pl.BlockSpec((BN, F), lambda i: (i, 0))
    t = pl.BlockSpec((128, 128), lambda i: (0, 0))
    return pl.pallas_call(
        _out_body,
        grid=(NB,),
        in_specs=[n, w, b, w, b,
                  pl.BlockSpec((F, 128), lambda i: (0, 0)),
                  pl.BlockSpec((BN, 1), lambda i: (i, 0)), t, t],
        out_specs=pl.BlockSpec((BN, 128), lambda i: (i, 0)),
        out_shape=jax.ShapeDtypeStruct((N, 128), jnp.float32),
    )(x, W1, b1, W2, b2, outw_pad, zcol, scale_b, shift_b)


def kernel(R, params, Z, BN_edge_index):
    src = BN_edge_index[0]
    dst = BN_edge_index[1]
    zcol = Z.reshape(N, 1)

    r128 = jnp.pad(R, ((0, 0), (0, RD - 3)))
    emb_pad = jnp.zeros((128, F), jnp.float32).at[:95].set(params['embedding'])
    centers = params['centers'].reshape(1, K)
    widths = params['widths'].reshape(1, K)
    cutoff = params['cutoff'].reshape(1, 1)
    zeros_stripe = jnp.zeros((ZR, HALF), jnp.float32)

    gatherR = _sc_gather(RD)
    gatherF = _sc_gather(F)
    scat0 = _sc_scatter_half(0)
    scat1 = _sc_scatter_half(HALF)

    rsrc = gatherR(r128, src)
    rdst = gatherR(r128, dst)
    rbf = _tc_rbf(rsrc, rdst, centers, widths, cutoff)

    x = _tc_emb(zcol, emb_pad)
    for m in params['modules']:
        xj, xi = _tc_nodeA(x, m['Wj'], m['bj'].reshape(1, F),
                           m['Wi'], m['bi'].reshape(1, F))
        xjsrc = gatherF(xj, src)
        msg = _tc_msg(rbf, xjsrc, m['G'])
        a0 = scat0(msg, dst, zeros_stripe)
        a1 = scat1(msg, dst, zeros_stripe)
        agg = jnp.concatenate([a0[:N], a1[:N]], axis=1)
        rb = m['int_res'][0]
        ab = m['at_res'][0]
        x = _tc_nodeF(x, xi, agg,
                      rb['W1'], rb['b1'].reshape(1, F),
                      rb['W2'], rb['b2'].reshape(1, F),
                      m['Wo'], m['bo'].reshape(1, F), m['u'].reshape(1, F),
                      ab['W1'], ab['b1'].reshape(1, F),
                      ab['W2'], ab['b2'].reshape(1, F))

    ob = params['out_res'][0]
    outw_pad = jnp.zeros((F, 128), jnp.float32).at[:, :1].set(params['out_W'])
    scale_b = jnp.zeros((128, 128), jnp.float32).at[:95].set(
        jnp.broadcast_to(params['scale'], (95, 128)))
    shift_b = jnp.zeros((128, 128), jnp.float32).at[:95].set(
        jnp.broadcast_to(params['shift'], (95, 128)))
    outp = _tc_out(x, ob['W1'], ob['b1'].reshape(1, F),
                   ob['W2'], ob['b2'].reshape(1, F),
                   outw_pad, zcol, scale_b, shift_b)
    return outp[:, :1]


# speedup vs baseline: 2.0856x; 1.3963x over previous
import functools
import math

import jax
import jax.numpy as jnp
from jax import lax
from jax.experimental import pallas as pl
from jax.experimental.pallas import tpu as pltpu
from jax.experimental.pallas import tpu_sc as plsc

N = 10000
E = 160000
F = 256
K = 64
LOG2 = math.log(2.0)

NB = 10
BN = N // NB
EB = 80
BE = E // EB

NW = 32
GCH = 40
GRP = 5
SCH = 80
SGR = 5
HALF = F // 2


def _ssp(x):
    return jax.nn.softplus(x) - LOG2


def _sc_gather(D):
    per_w = E // NW
    iters = per_w // GCH
    mesh = plsc.VectorSubcoreMesh(core_axis_name="c", subcore_axis_name="s")

    @functools.partial(
        pl.kernel, mesh=mesh,
        out_type=jax.ShapeDtypeStruct((E, D), jnp.float32),
        scratch_types=[
            pltpu.VMEM((per_w,), jnp.int32),
            pltpu.VMEM((GRP * GCH, D), jnp.float32),
            pltpu.SemaphoreType.DMA,
        ],
    )
    def k(table_hbm, idx_hbm, out_hbm, idx_v, rows_v, sem):
        wid = lax.axis_index("s") * 2 + lax.axis_index("c")
        base = wid * per_w
        pltpu.sync_copy(idx_hbm.at[pl.ds(base, per_w)], idx_v)

        def body(i, carry):
            off = i * GRP * GCH
            cps = [
                pltpu.async_copy(
                    table_hbm.at[idx_v.at[pl.ds(off + b * GCH, GCH)]],
                    rows_v.at[pl.ds(b * GCH, GCH)], sem)
                for b in range(GRP)
            ]
            for cp in cps:
                cp.wait()
            pltpu.sync_copy(rows_v, out_hbm.at[pl.ds(base + off, GRP * GCH)])
            return carry

        lax.fori_loop(0, iters // GRP, body, 0)

    return k


NROW = 5120
NACC = NROW + 8
ZR = NROW // 16


def _sc_scatter_half(hoff):
    per_s = E // 16
    iters = per_s // SCH
    mesh = plsc.VectorSubcoreMesh(core_axis_name="c", subcore_axis_name="s")

    @functools.partial(
        pl.kernel, mesh=mesh,
        out_type=jax.ShapeDtypeStruct((2 * NROW, HALF), jnp.float32),
        scratch_types=[
            pltpu.VMEM((E // 16,), jnp.int32),
            pltpu.VMEM((SGR * SCH, HALF), jnp.float32),
            pltpu.VMEM_SHARED((NACC, HALF), jnp.float32),
            pltpu.SemaphoreType.DMA,
        ] + [pltpu.VMEM((SCH,), jnp.int32) for _ in range(SGR)],
    )
    def k(msg_hbm, dst_hbm, zero_hbm, agg_hbm, idx_v, rows_v,
          shared, sem, *idx_sc):
        c = lax.axis_index("c")
        s = lax.axis_index("s")
        nbase = c * NROW
        pltpu.sync_copy(zero_hbm, rows_v.at[pl.ds(0, ZR // 2)])
        for h in range(2):
            pltpu.sync_copy(rows_v.at[pl.ds(0, ZR // 2)],
                            shared.at[pl.ds(s * ZR + h * (ZR // 2), ZR // 2)])
        pltpu.sync_copy(dst_hbm.at[pl.ds(s * per_s, per_s)], idx_v)
        plsc.subcore_barrier()

        def body(i, carry):
            off = i * SGR * SCH
            pltpu.sync_copy(
                msg_hbm.at[pl.ds(s * per_s + off, SGR * SCH),
                           pl.ds(hoff, HALF)], rows_v)
            cps = []
            for b in range(SGR):
                o2 = off + b * SCH
                for j in range(SCH // 16):
                    v = idx_v[pl.ds(o2 + j * 16, 16)]
                    rel = v - nbase
                    ok = (rel >= 0) & (rel < NROW)
                    idx_sc[b][pl.ds(j * 16, 16)] = jnp.where(ok, rel, NROW)
                cps.append(pltpu.async_copy(
                    rows_v.at[pl.ds(b * SCH, SCH)], shared.at[idx_sc[b]],
                    sem, add=True))
            for cp in cps:
                cp.wait()
            return carry

        lax.fori_loop(0, iters // SGR, body, 0)
        plsc.subcore_barrier()
        for h in range(2):
            o3 = s * ZR + h * (ZR // 2)
            pltpu.sync_copy(shared.at[pl.ds(o3, ZR // 2)],
                            rows_v.at[pl.ds(0, ZR // 2)])
            pltpu.sync_copy(rows_v.at[pl.ds(0, ZR // 2)],
                            agg_hbm.at[pl.ds(nbase + o3, ZR // 2)])

    return k


def _emb_body(z_ref, emb_ref, o_ref):
    col = lax.broadcasted_iota(jnp.int32, (BN, 128), 1)
    oh = (col == z_ref[:, :1]).astype(jnp.float32)
    o_ref[...] = jnp.dot(oh, emb_ref[...], preferred_element_type=jnp.float32)


def _tc_emb(zcol, emb_pad):
    return pl.pallas_call(
        _emb_body,
        grid=(NB,),
        in_specs=[pl.BlockSpec((BN, 1), lambda i: (i, 0)),
                  pl.BlockSpec((128, F), lambda i: (0, 0))],
        out_specs=pl.BlockSpec((BN, F), lambda i: (i, 0)),
        out_shape=jax.ShapeDtypeStruct((N, F), jnp.float32),
    )(zcol, emb_pad)


def _nodeA_body(x_ref, wj_ref, bj_ref, wi_ref, bi_ref, xj_ref, xi_ref):
    xa = _ssp(x_ref[...])
    xj_ref[...] = jnp.dot(xa, wj_ref[...],
                          preferred_element_type=jnp.float32) + bj_ref[...]
    xi_ref[...] = jnp.dot(xa, wi_ref[...],
                          preferred_element_type=jnp.float32) + bi_ref[...]


def _tc_nodeA(x, Wj, bj, Wi, bi):
    w = pl.BlockSpec((F, F), lambda i: (0, 0))
    b = pl.BlockSpec((1, F), lambda i: (0, 0))
    n = pl.BlockSpec((BN, F), lambda i: (i, 0))
    return pl.pallas_call(
        _nodeA_body,
        grid=(NB,),
        in_specs=[n, w, b, w, b],
        out_specs=[n, n],
        out_shape=[jax.ShapeDtypeStruct((N, F), jnp.float32),
                   jax.ShapeDtypeStruct((N, F), jnp.float32)],
    )(x, Wj, bj, Wi, bi)


RD = 128


def _rbf_body(rs_ref, rd_ref, c_ref, w_ref, co_ref, o_ref):
    diff = rs_ref[...] - rd_ref[...]
    d2 = jnp.sum(diff * diff, axis=1, keepdims=True)
    d = jnp.sqrt(d2 + 1e-10)
    cutoff = co_ref[0, 0]
    r = d / cutoff
    phi = 1.0 - 6.0 * r ** 5 + 15.0 * r ** 4 - 10.0 * r ** 3
    phi = jnp.where(d < cutoff, phi, jnp.zeros_like(phi))
    o_ref[...] = phi * jnp.exp(-w_ref[...] * (jnp.exp(-d) - c_ref[...]) ** 2)


def _tc_rbf(rsrc, rdst, centers, widths, cutoff):
    e = pl.BlockSpec((BE, RD), lambda i: (i, 0))
    v = pl.BlockSpec((1, K), lambda i: (0, 0))
    return pl.pallas_call(
        _rbf_body,
        grid=(EB,),
        in_specs=[e, e, v, v, pl.BlockSpec((1, 1), lambda i: (0, 0))],
        out_specs=pl.BlockSpec((BE, K), lambda i: (i, 0)),
        out_shape=jax.ShapeDtypeStruct((E, K), jnp.float32),
    )(rsrc, rdst, centers, widths, cutoff)


def _msg_body(rbf_ref, xjs_ref, g_ref, o_ref):
    g = jnp.dot(rbf_ref[...], g_ref[...], preferred_element_type=jnp.float32)
    o_ref[...] = g * xjs_ref[...]


def _tc_msg(rbf, xjsrc, G):
    return pl.pallas_call(
        _msg_body,
        grid=(EB,),
        in_specs=[pl.BlockSpec((BE, K), lambda i: (i, 0)),
                  pl.BlockSpec((BE, F), lambda i: (i, 0)),
                  pl.BlockSpec((K, F), lambda i: (0, 0))],
        out_specs=pl.BlockSpec((BE, F), lambda i: (i, 0)),
        out_shape=jax.ShapeDtypeStruct((E, F), jnp.float32),
    )(rbf, xjsrc, G)


def _res(x, w1, b1, w2, b2):
    y = jnp.dot(_ssp(x), w1, preferred_element_type=jnp.float32) + b1
    y = jnp.dot(_ssp(y), w2, preferred_element_type=jnp.float32) + b2
    return x + y


def _nodeF_body(x_ref, xi_ref, agg_ref, w1, b1, w2, b2, wo, bo, u,
                a1, ab1, a2, ab2, o_ref):
    mi = xi_ref[...] + agg_ref[...]
    mi = _res(mi, w1[...], b1[...], w2[...], b2[...])
    xn = u[...] * x_ref[...] + jnp.dot(
        _ssp(mi), wo[...], preferred_element_type=jnp.float32) + bo[...]
    o_ref[...] = _res(xn, a1[...], ab1[...], a2[...], ab2[...])


def _tc_nodeF(x, xi, agg, W1, b1, W2, b2, Wo, bo, u, aW1, ab1, aW2, ab2):
    w = pl.BlockSpec((F, F), lambda i: (0, 0))
    b = pl.BlockSpec((1, F), lambda i: (0, 0))
    n = pl.BlockSpec((BN, F), lambda i: (i, 0))
    return pl.pallas_call(
        _nodeF_body,
        grid=(NB,),
        in_specs=[n, n, n, w, b, w, b, w, b, b, w, b, w, b],
        out_specs=n,
        out_shape=jax.ShapeDtypeStruct((N, F), jnp.float32),
    )(x, xi, agg, W1, b1, W2, b2, Wo, bo, u, aW1, ab1, aW2, ab2)


def _out_body(x_ref, w1, b1, w2, b2, ow, z_ref, sc_ref, sh_ref, o_ref):
    o = _res(x_ref[...], w1[...], b1[...], w2[...], b2[...])
    val = jnp.dot(_ssp(o), ow[...], preferred_element_type=jnp.float32)
    col = lax.broadcasted_iota(jnp.int32, (BN, 128), 1)
    oh = (col == z_ref[:, :1]).astype(jnp.float32)
    scale = jnp.dot(oh, sc_ref[...], preferred_element_type=jnp.float32)
    shift = jnp.dot(oh, sh_ref[...], preferred_element_type=jnp.float32)
    o_ref[...] = scale * val + shift


def _tc_out(x, W1, b1, W2, b2, outw_pad, zcol, scale_b, shift_b):
    w = pl.BlockSpec((F, F), lambda i: (0, 0))
    b = pl.BlockSpec((1, F), lambda i: (0, 0))
    n = pl.BlockSpec((BN, F), lambda i: (i, 0))
    t = pl.BlockSpec((128, 128), lambda i: (0, 0))
    return pl.pallas_call(
        _out_body,
        grid=(NB,),
        in_specs=[n, w, b, w, b,
                  pl.BlockSpec((F, 128), lambda i: (0, 0)),
                  pl.BlockSpec((BN, 1), lambda i: (i, 0)), t, t],
        out_specs=pl.BlockSpec((BN, 128), lambda i: (i, 0)),
        out_shape=jax.ShapeDtypeStruct((N, 128), jnp.float32),
    )(x, W1, b1, W2, b2, outw_pad, zcol, scale_b, shift_b)


def kernel(R, params, Z, BN_edge_index):
    src = BN_edge_index[0]
    dst = BN_edge_index[1]
    zcol = Z.reshape(N, 1)

    r128 = jnp.pad(R, ((0, 0), (0, RD - 3)))
    emb_pad = jnp.zeros((128, F), jnp.float32).at[:95].set(params['embedding'])
    centers = params['centers'].reshape(1, K)
    widths = params['widths'].reshape(1, K)
    cutoff = params['cutoff'].reshape(1, 1)
    zeros_stripe = jnp.zeros((ZR // 2, HALF), jnp.float32)

    gatherR = _sc_gather(RD)
    gatherF = _sc_gather(F)
    scat0 = _sc_scatter_half(0)
    scat1 = _sc_scatter_half(HALF)

    rsrc = gatherR(r128, src)
    rdst = gatherR(r128, dst)
    rbf = _tc_rbf(rsrc, rdst, centers, widths, cutoff)

    x = _tc_emb(zcol, emb_pad)
    for m in params['modules']:
        xj, xi = _tc_nodeA(x, m['Wj'], m['bj'].reshape(1, F),
                           m['Wi'], m['bi'].reshape(1, F))
        xjsrc = gatherF(xj, src)
        msg = _tc_msg(rbf, xjsrc, m['G'])
        a0 = scat0(msg, dst, zeros_stripe)
        a1 = scat1(msg, dst, zeros_stripe)
        agg = jnp.concatenate([a0[:N], a1[:N]], axis=1)
        rb = m['int_res'][0]
        ab = m['at_res'][0]
        x = _tc_nodeF(x, xi, agg,
                      rb['W1'], rb['b1'].reshape(1, F),
                      rb['W2'], rb['b2'].reshape(1, F),
                      m['Wo'], m['bo'].reshape(1, F), m['u'].reshape(1, F),
                      ab['W1'], ab['b1'].reshape(1, F),
                      ab['W2'], ab['b2'].reshape(1, F))

    ob = params['out_res'][0]
    outw_pad = jnp.zeros((F, 128), jnp.float32).at[:, :1].set(params['out_W'])
    scale_b = jnp.zeros((128, 128), jnp.float32).at[:95].set(
        jnp.broadcast_to(params['scale'], (95, 128)))
    shift_b = jnp.zeros((128, 128), jnp.float32).at[:95].set(
        jnp.broadcast_to(params['shift'], (95, 128)))
    outp = _tc_out(x, ob['W1'], ob['b1'].reshape(1, F),
                   ob['W2'], ob['b2'].reshape(1, F),
                   outw_pad, zcol, scale_b, shift_b)
    return outp[:, :1]


# speedup vs baseline: 2.0989x; 1.0064x over previous
import functools
import math

import jax
import jax.numpy as jnp
from jax import lax
from jax.experimental import pallas as pl
from jax.experimental.pallas import tpu as pltpu
from jax.experimental.pallas import tpu_sc as plsc

N = 10000
E = 160000
F = 256
K = 64
LOG2 = math.log(2.0)

NB = 10
BN = N // NB
EB = 80
BE = E // EB

NW = 32
GCH = 40
GRP = 5
SCH = 80
SGR = 5
HALF = F // 2


def _ssp(x):
    return jax.nn.softplus(x) - LOG2


def _sc_gather(D, NE=E):
    per_w = NE // NW
    iters = per_w // GCH
    mesh = plsc.VectorSubcoreMesh(core_axis_name="c", subcore_axis_name="s")

    @functools.partial(
        pl.kernel, mesh=mesh,
        out_type=jax.ShapeDtypeStruct((NE, D), jnp.float32),
        scratch_types=[
            pltpu.VMEM((per_w,), jnp.int32),
            pltpu.VMEM((GRP * GCH, D), jnp.float32),
            pltpu.SemaphoreType.DMA,
        ],
    )
    def k(table_hbm, idx_hbm, out_hbm, idx_v, rows_v, sem):
        wid = lax.axis_index("s") * 2 + lax.axis_index("c")
        base = wid * per_w
        pltpu.sync_copy(idx_hbm.at[pl.ds(base, per_w)], idx_v)

        def body(i, carry):
            off = i * GRP * GCH
            cps = [
                pltpu.async_copy(
                    table_hbm.at[idx_v.at[pl.ds(off + b * GCH, GCH)]],
                    rows_v.at[pl.ds(b * GCH, GCH)], sem)
                for b in range(GRP)
            ]
            for cp in cps:
                cp.wait()
            pltpu.sync_copy(rows_v, out_hbm.at[pl.ds(base + off, GRP * GCH)])
            return carry

        lax.fori_loop(0, iters // GRP, body, 0)

    return k


NROW = 5120
NACC = NROW + 8
ZR = NROW // 16


def _sc_scatter_half(hoff):
    per_s = E // 16
    iters = per_s // SCH
    mesh = plsc.VectorSubcoreMesh(core_axis_name="c", subcore_axis_name="s")

    @functools.partial(
        pl.kernel, mesh=mesh,
        out_type=jax.ShapeDtypeStruct((2 * NROW, HALF), jnp.float32),
        scratch_types=[
            pltpu.VMEM((E // 16,), jnp.int32),
            pltpu.VMEM((SGR * SCH, HALF), jnp.float32),
            pltpu.VMEM_SHARED((NACC, HALF), jnp.float32),
            pltpu.SemaphoreType.DMA,
        ] + [pltpu.VMEM((SCH,), jnp.int32) for _ in range(SGR)],
    )
    def k(msg_hbm, dst_hbm, zero_hbm, agg_hbm, idx_v, rows_v,
          shared, sem, *idx_sc):
        c = lax.axis_index("c")
        s = lax.axis_index("s")
        nbase = c * NROW
        pltpu.sync_copy(zero_hbm, rows_v.at[pl.ds(0, ZR // 2)])
        for h in range(2):
            pltpu.sync_copy(rows_v.at[pl.ds(0, ZR // 2)],
                            shared.at[pl.ds(s * ZR + h * (ZR // 2), ZR // 2)])
        pltpu.sync_copy(dst_hbm.at[pl.ds(s * per_s, per_s)], idx_v)
        plsc.subcore_barrier()

        def body(i, carry):
            off = i * SGR * SCH
            pltpu.sync_copy(
                msg_hbm.at[pl.ds(s * per_s + off, SGR * SCH),
                           pl.ds(hoff, HALF)], rows_v)
            cps = []
            for b in range(SGR):
                o2 = off + b * SCH
                for j in range(SCH // 16):
                    v = idx_v[pl.ds(o2 + j * 16, 16)]
                    rel = v - nbase
                    ok = (rel >= 0) & (rel < NROW)
                    idx_sc[b][pl.ds(j * 16, 16)] = jnp.where(ok, rel, NROW)
                cps.append(pltpu.async_copy(
                    rows_v.at[pl.ds(b * SCH, SCH)], shared.at[idx_sc[b]],
                    sem, add=True))
            for cp in cps:
                cp.wait()
            return carry

        lax.fori_loop(0, iters // SGR, body, 0)
        plsc.subcore_barrier()
        for h in range(2):
            o3 = s * ZR + h * (ZR // 2)
            pltpu.sync_copy(shared.at[pl.ds(o3, ZR // 2)],
                            rows_v.at[pl.ds(0, ZR // 2)])
            pltpu.sync_copy(rows_v.at[pl.ds(0, ZR // 2)],
                            agg_hbm.at[pl.ds(nbase + o3, ZR // 2)])

    return k


def _emb_body(z_ref, emb_ref, o_ref):
    col = lax.broadcasted_iota(jnp.int32, (BN, 128), 1)
    oh = (col == z_ref[:, :1]).astype(jnp.float32)
    o_ref[...] = jnp.dot(oh, emb_ref[...], preferred_element_type=jnp.float32)


def _tc_emb(zcol, emb_pad):
    return pl.pallas_call(
        _emb_body,
        grid=(NB,),
        in_specs=[pl.BlockSpec((BN, 1), lambda i: (i, 0)),
                  pl.BlockSpec((128, F), lambda i: (0, 0))],
        out_specs=pl.BlockSpec((BN, F), lambda i: (i, 0)),
        out_shape=jax.ShapeDtypeStruct((N, F), jnp.float32),
    )(zcol, emb_pad)


def _nodeA_body(x_ref, wj_ref, bj_ref, wi_ref, bi_ref, xj_ref, xi_ref):
    xa = _ssp(x_ref[...])
    xj_ref[...] = jnp.dot(xa, wj_ref[...],
                          preferred_element_type=jnp.float32) + bj_ref[...]
    xi_ref[...] = jnp.dot(xa, wi_ref[...],
                          preferred_element_type=jnp.float32) + bi_ref[...]


def _tc_nodeA(x, Wj, bj, Wi, bi):
    w = pl.BlockSpec((F, F), lambda i: (0, 0))
    b = pl.BlockSpec((1, F), lambda i: (0, 0))
    n = pl.BlockSpec((BN, F), lambda i: (i, 0))
    return pl.pallas_call(
        _nodeA_body,
        grid=(NB,),
        in_specs=[n, w, b, w, b],
        out_specs=[n, n],
        out_shape=[jax.ShapeDtypeStruct((N, F), jnp.float32),
                   jax.ShapeDtypeStruct((N, F), jnp.float32)],
    )(x, Wj, bj, Wi, bi)


RD = 128


def _rbf_body(rs_ref, rd_ref, c_ref, w_ref, co_ref, o_ref):
    diff = rs_ref[...] - rd_ref[...]
    d2 = jnp.sum(diff * diff, axis=1, keepdims=True)
    d = jnp.sqrt(d2 + 1e-10)
    cutoff = co_ref[0, 0]
    r = d / cutoff
    phi = 1.0 - 6.0 * r ** 5 + 15.0 * r ** 4 - 10.0 * r ** 3
    phi = jnp.where(d < cutoff, phi, jnp.zeros_like(phi))
    o_ref[...] = phi * jnp.exp(-w_ref[...] * (jnp.exp(-d) - c_ref[...]) ** 2)


def _tc_rbf(rsrc, rdst, centers, widths, cutoff):
    e = pl.BlockSpec((BE, RD), lambda i: (i, 0))
    e2 = pl.BlockSpec((BE, RD), lambda i: (i + EB, 0))
    v = pl.BlockSpec((1, K), lambda i: (0, 0))
    return pl.pallas_call(
        _rbf_body,
        grid=(EB,),
        in_specs=[e, e2, v, v, pl.BlockSpec((1, 1), lambda i: (0, 0))],
        out_specs=pl.BlockSpec((BE, K), lambda i: (i, 0)),
        out_shape=jax.ShapeDtypeStruct((E, K), jnp.float32),
    )(rsrc, rdst, centers, widths, cutoff)


def _msg_body(rbf_ref, xjs_ref, g_ref, o_ref):
    g = jnp.dot(rbf_ref[...], g_ref[...], preferred_element_type=jnp.float32)
    o_ref[...] = g * xjs_ref[...]


def _tc_msg(rbf, xjsrc, G):
    return pl.pallas_call(
        _msg_body,
        grid=(EB,),
        in_specs=[pl.BlockSpec((BE, K), lambda i: (i, 0)),
                  pl.BlockSpec((BE, F), lambda i: (i, 0)),
                  pl.BlockSpec((K, F), lambda i: (0, 0))],
        out_specs=pl.BlockSpec((BE, F), lambda i: (i, 0)),
        out_shape=jax.ShapeDtypeStruct((E, F), jnp.float32),
    )(rbf, xjsrc, G)


def _res(x, w1, b1, w2, b2):
    y = jnp.dot(_ssp(x), w1, preferred_element_type=jnp.float32) + b1
    y = jnp.dot(_ssp(y), w2, preferred_element_type=jnp.float32) + b2
    return x + y


def _nodeF_body(x_ref, xi_ref, agg_ref, w1, b1, w2, b2, wo, bo, u,
                a1, ab1, a2, ab2, o_ref):
    mi = xi_ref[...] + agg_ref[...]
    mi = _res(mi, w1[...], b1[...], w2[...], b2[...])
    xn = u[...] * x_ref[...] + jnp.dot(
        _ssp(mi), wo[...], preferred_element_type=jnp.float32) + bo[...]
    o_ref[...] = _res(xn, a1[...], ab1[...], a2[...], ab2[...])


def _tc_nodeF(x, xi, agg, W1, b1, W2, b2, Wo, bo, u, aW1, ab1, aW2, ab2):
    w = pl.BlockSpec((F, F), lambda i: (0, 0))
    b = pl.BlockSpec((1, F), lambda i: (0, 0))
    n = pl.BlockSpec((BN, F), lambda i: (i, 0))
    return pl.pallas_call(
        _nodeF_body,
        grid=(NB,),
        in_specs=[n, n, n, w, b, w, b, w, b, b, w, b, w, b],
        out_specs=n,
        out_shape=jax.ShapeDtypeStruct((N, F), jnp.float32),
    )(x, xi, agg, W1, b1, W2, b2, Wo, bo, u, aW1, ab1, aW2, ab2)


def _out_body(x_ref, w1, b1, w2, b2, ow, z_ref, sc_ref, sh_ref, o_ref):
    o = _res(x_ref[...], w1[...], b1[...], w2[...], b2[...])
    val = jnp.dot(_ssp(o), ow[...], preferred_element_type=jnp.float32)
    col = lax.broadcasted_iota(jnp.int32, (BN, 128), 1)
    oh = (col == z_ref[:, :1]).astype(jnp.float32)
    scale = jnp.dot(oh, sc_ref[...], preferred_element_type=jnp.float32)
    shift = jnp.dot(oh, sh_ref[...], preferred_element_type=jnp.float32)
    o_ref[...] = scale * val + shift


def _tc_out(x, W1, b1, W2, b2, outw_pad, zcol, scale_b, shift_b):
    w = pl.BlockSpec((F, F), lambda i: (0, 0))
    b = pl.BlockSpec((1, F), lambda i: (0, 0))
    n = pl.BlockSpec((BN, F), lambda i: (i, 0))
    t = pl.BlockSpec((128, 128), lambda i: (0, 0))
    return pl.pallas_call(
        _out_body,
        grid=(NB,),
        in_specs=[n, w, b, w, b,
                  pl.BlockSpec((F, 128), lambda i: (0, 0)),
                  pl.BlockSpec((BN, 1), lambda i: (i, 0)), t, t],
        out_specs=pl.BlockSpec((BN, 128), lambda i: (i, 0)),
        out_shape=jax.ShapeDtypeStruct((N, 128), jnp.float32),
    )(x, W1, b1, W2, b2, outw_pad, zcol, scale_b, shift_b)


def kernel(R, params, Z, BN_edge_index):
    src = BN_edge_index[0]
    dst = BN_edge_index[1]
    zcol = Z.reshape(N, 1)

    r128 = jnp.pad(R, ((0, 0), (0, RD - 3)))
    emb_pad = jnp.zeros((128, F), jnp.float32).at[:95].set(params['embedding'])
    centers = params['centers'].reshape(1, K)
    widths = params['widths'].reshape(1, K)
    cutoff = params['cutoff'].reshape(1, 1)
    zeros_stripe = jnp.zeros((ZR // 2, HALF), jnp.float32)

    gatherR = _sc_gather(RD, 2 * E)
    gatherF = _sc_gather(F)
    scat0 = _sc_scatter_half(0)
    scat1 = _sc_scatter_half(HALF)

    rall = gatherR(r128, jnp.concatenate([src, dst]))
    rbf = _tc_rbf(rall, rall, centers, widths, cutoff)

    x = _tc_emb(zcol, emb_pad)
    for m in params['modules']:
        xj, xi = _tc_nodeA(x, m['Wj'], m['bj'].reshape(1, F),
                           m['Wi'], m['bi'].reshape(1, F))
        xjsrc = gatherF(xj, src)
        msg = _tc_msg(rbf, xjsrc, m['G'])
        a0 = scat0(msg, dst, zeros_stripe)
        a1 = scat1(msg, dst, zeros_stripe)
        agg = jnp.concatenate([a0[:N], a1[:N]], axis=1)
        rb = m['int_res'][0]
        ab = m['at_res'][0]
        x = _tc_nodeF(x, xi, agg,
                      rb['W1'], rb['b1'].reshape(1, F),
                      rb['W2'], rb['b2'].reshape(1, F),
                      m['Wo'], m['bo'].reshape(1, F), m['u'].reshape(1, F),
                      ab['W1'], ab['b1'].reshape(1, F),
                      ab['W2'], ab['b2'].reshape(1, F))

    ob = params['out_res'][0]
    outw_pad = jnp.zeros((F, 128), jnp.float32).at[:, :1].set(params['out_W'])
    scale_b = jnp.zeros((128, 128), jnp.float32).at[:95].set(
        jnp.broadcast_to(params['scale'], (95, 128)))
    shift_b = jnp.zeros((128, 128), jnp.float32).at[:95].set(
        jnp.broadcast_to(params['shift'], (95, 128)))
    outp = _tc_out(x, ob['W1'], ob['b1'].reshape(1, F),
                   ob['W2'], ob['b2'].reshape(1, F),
                   outw_pad, zcol, scale_b, shift_b)
    return outp[:, :1]
